# baseline (speedup 1.0000x reference)
import functools

import jax
import jax.numpy as jnp
from jax import lax
from jax.experimental import pallas as pl
from jax.experimental.pallas import tpu as pltpu
from jax.experimental.pallas import tpu_sc as plsc

_N_E = 512
_E_DIM = 32
_BETA = 0.25
_N_TOK = 131072


_BZ = 2048
_NB = _N_TOK // _BZ


def _rowsum32(t):
    u = ((t[:, 0:8] + t[:, 8:16]) + t[:, 16:24]) + t[:, 24:32]
    v = u[:, 0:4] + u[:, 4:8]
    w = v[:, 0:2] + v[:, 2:4]
    return w[:, 0:1] + w[:, 1:2]


def _tc_body(z_ref, emb_ref, idx_ref, acc_ref, mm_s, en_s, cv_s, cr_s):
    z = z_ref[...]
    emb = emb_ref[...]
    mm_s[...] = lax.dot_general(emb, z + z, (((1,), (1,)), ((), ())),
                                preferred_element_type=jnp.float32,
                                precision="default")
    znT = _rowsum32(z * z).reshape(1, _BZ)
    zn8 = jnp.broadcast_to(znT, (8, _BZ))

    @pl.when(pl.program_id(0) == 0)
    def _():
        en_s[...] = jnp.broadcast_to(_rowsum32(emb * emb), (_N_E, _BZ))
        acc_ref[...] = jnp.zeros_like(acc_ref)

    cv_s[...] = (zn8 + en_s[0:8, :]) - mm_s[0:8, :]
    cr_s[...] = jnp.zeros((8, _BZ), jnp.int32)
    for r in range(1, _N_E // 8):
        blk = (zn8 + en_s[8 * r:8 * r + 8, :]) - mm_s[8 * r:8 * r + 8, :]
        cur = cv_s[...]
        msk = blk < cur
        cv_s[...] = jnp.where(msk, blk, cur)
        cr_s[...] = jnp.where(msk, r, cr_s[...])
    riota = lax.broadcasted_iota(jnp.int32, (8, _BZ), 0)
    cur_v = cv_s[...]
    cur_i = cr_s[...] * 8 + riota
    for sh in (4, 2, 1):
        av, bv = cur_v[0:sh], cur_v[sh:2 * sh]
        ai, bi = cur_i[0:sh], cur_i[sh:2 * sh]
        take = (bv < av) | ((bv == av) & (bi < ai))
        cur_v = jnp.where(take, bv, av)
        cur_i = jnp.where(take, bi, ai)
    idx_ref[...] = cur_i.reshape(_BZ)
    acc_ref[...] += jnp.sum(cur_v).reshape(1, 1)


_tc_call = pl.pallas_call(
    _tc_body,
    grid=(_NB,),
    in_specs=[
        pl.BlockSpec((_BZ, _E_DIM), lambda i: (i, 0)),
        pl.BlockSpec((_N_E, _E_DIM), lambda i: (0, 0)),
    ],
    out_specs=[
        pl.BlockSpec((_BZ,), lambda i: (i,)),
        pl.BlockSpec((1, 1), lambda i: (0, 0)),
    ],
    out_shape=[
        jax.ShapeDtypeStruct((_N_TOK,), jnp.int32),
        jax.ShapeDtypeStruct((1, 1), jnp.float32),
    ],
    scratch_shapes=[
        pltpu.VMEM((_N_E, _BZ), jnp.float32),
        pltpu.VMEM((_N_E, _BZ), jnp.float32),
        pltpu.VMEM((8, _BZ), jnp.float32),
        pltpu.VMEM((8, _BZ), jnp.int32),
    ],
)


_NC = 2
_NS = 16
_NW = _NC * _NS
_BPW = _N_TOK // _NW
_CH = 1024
_NCH = _BPW // _CH


def _sc_gather_body(emb_hbm, idx_hbm, out_hbm, idx_v, buf_a, buf_b, sem_a,
                    sem_b, osem):
    wid = lax.axis_index("s") * _NC + lax.axis_index("c")
    base = wid * _BPW
    bufs = (buf_a, buf_b)
    sems = (sem_a, sem_b)
    pltpu.sync_copy(idx_hbm.at[pl.ds(base, _BPW)], idx_v)
    gathers = [pltpu.async_copy(emb_hbm.at[idx_v.at[pl.ds(0, _CH)]],
                                bufs[0], sems[0])]
    writes = [None, None]
    for c in range(_NCH):
        b = c % 2
        nb = (c + 1) % 2
        if c + 1 < _NCH:
            off = (c + 1) * _CH
            if writes[nb] is not None:
                writes[nb].wait()
                writes[nb] = None
            gathers.append(
                pltpu.async_copy(emb_hbm.at[idx_v.at[pl.ds(off, _CH)]],
                                 bufs[nb], sems[nb]))
        gathers[c].wait()
        writes[b] = pltpu.async_copy(bufs[b],
                                     out_hbm.at[pl.ds(base + c * _CH, _CH)],
                                     osem)
    for w in writes:
        if w is not None:
            w.wait()


_sc_gather = pl.kernel(
    _sc_gather_body,
    mesh=plsc.VectorSubcoreMesh(core_axis_name="c", subcore_axis_name="s"),
    out_type=jax.ShapeDtypeStruct((_N_TOK, _E_DIM), jnp.float32),
    compiler_params=pltpu.CompilerParams(use_tc_tiling_on_sc=False),
    scratch_types=[
        pltpu.VMEM((_BPW,), jnp.int32),
        pltpu.VMEM((_CH, _E_DIM), jnp.float32),
        pltpu.VMEM((_CH, _E_DIM), jnp.float32),
        pltpu.SemaphoreType.DMA,
        pltpu.SemaphoreType.DMA,
        pltpu.SemaphoreType.DMA,
    ],
)


def kernel(z, emb):
    idx, acc = _tc_call(z, emb)
    z_q = _sc_gather(emb, idx)
    loss = acc[0, 0] * ((1.0 + _BETA) / (_N_TOK * _E_DIM))
    return (z_q, loss)


# speedup vs baseline: 110.7702x; 110.7702x over previous
import functools

import jax
import jax.numpy as jnp
from jax import lax
from jax.experimental import pallas as pl
from jax.experimental.pallas import tpu as pltpu
from jax.experimental.pallas import tpu_sc as plsc

_N_E = 512
_E_DIM = 32
_BETA = 0.25
_N_TOK = 131072


_BZ = 2048
_NB = _N_TOK // _BZ


def _rowsum32(t):
    u = ((t[:, 0:8] + t[:, 8:16]) + t[:, 16:24]) + t[:, 24:32]
    v = u[:, 0:4] + u[:, 4:8]
    w = v[:, 0:2] + v[:, 2:4]
    return w[:, 0:1] + w[:, 1:2]


def _tc_body(z_ref, emb_ref, idx_ref, acc_ref):
    z = z_ref[...]
    emb = emb_ref[...]
    mm = lax.dot_general(z, emb, (((1,), (1,)), ((), ())),
                         preferred_element_type=jnp.float32,
                         precision="default")
    zn = _rowsum32(z * z)
    en = _rowsum32(emb * emb)[:, 0]
    d = (zn + en) - 2.0 * mm
    m = jnp.min(d, axis=1, keepdims=True)
    col = lax.broadcasted_iota(jnp.int32, (_BZ, _N_E), 1)
    idx = jnp.min(jnp.where(d == m, col, _N_E), axis=1)
    idx_ref[0, 0, :] = idx.astype(jnp.int32)
    acc_ref[0] = jnp.broadcast_to(jnp.sum(m).reshape(1, 1), (8, 128))


_tc_call = pl.pallas_call(
    _tc_body,
    grid=(_NB,),
    in_specs=[
        pl.BlockSpec((_BZ, _E_DIM), lambda i: (i, 0)),
        pl.BlockSpec((_N_E, _E_DIM), lambda i: (0, 0)),
    ],
    out_specs=[
        pl.BlockSpec((1, 1, _BZ), lambda i: (i, 0, 0)),
        pl.BlockSpec((1, 8, 128), lambda i: (i, 0, 0)),
    ],
    out_shape=[
        jax.ShapeDtypeStruct((_NB, 1, _BZ), jnp.int32),
        jax.ShapeDtypeStruct((_NB, 8, 128), jnp.float32),
    ],
    compiler_params=pltpu.CompilerParams(
        dimension_semantics=("parallel",)),
)


_NC = 2
_NS = 16
_NW = _NC * _NS
_BPW = _N_TOK // _NW
_CH = 1024
_NCH = _BPW // _CH


def _sc_gather_body(emb_hbm, idx_hbm, out_hbm, idx_v, buf_a, buf_b, sem_a,
                    sem_b, osem):
    wid = lax.axis_index("s") * _NC + lax.axis_index("c")
    base = wid * _BPW
    bufs = (buf_a, buf_b)
    sems = (sem_a, sem_b)
    pltpu.sync_copy(idx_hbm.at[pl.ds(base, _BPW)], idx_v)
    gathers = [pltpu.async_copy(emb_hbm.at[idx_v.at[pl.ds(0, _CH)]],
                                bufs[0], sems[0])]
    writes = [None, None]
    for c in range(_NCH):
        b = c % 2
        nb = (c + 1) % 2
        if c + 1 < _NCH:
            off = (c + 1) * _CH
            if writes[nb] is not None:
                writes[nb].wait()
                writes[nb] = None
            gathers.append(
                pltpu.async_copy(emb_hbm.at[idx_v.at[pl.ds(off, _CH)]],
                                 bufs[nb], sems[nb]))
        gathers[c].wait()
        writes[b] = pltpu.async_copy(bufs[b],
                                     out_hbm.at[pl.ds(base + c * _CH, _CH)],
                                     osem)
    for w in writes:
        if w is not None:
            w.wait()


_sc_gather = pl.kernel(
    _sc_gather_body,
    mesh=plsc.VectorSubcoreMesh(core_axis_name="c", subcore_axis_name="s"),
    out_type=jax.ShapeDtypeStruct((_N_TOK, _E_DIM), jnp.float32),
    compiler_params=pltpu.CompilerParams(use_tc_tiling_on_sc=False),
    scratch_types=[
        pltpu.VMEM((_BPW,), jnp.int32),
        pltpu.VMEM((_CH, _E_DIM), jnp.float32),
        pltpu.VMEM((_CH, _E_DIM), jnp.float32),
        pltpu.SemaphoreType.DMA,
        pltpu.SemaphoreType.DMA,
        pltpu.SemaphoreType.DMA,
    ],
)


def kernel(z, emb):
    idx3, acc = _tc_call(z, emb)
    idx = idx3.reshape(_N_TOK)
    z_q = _sc_gather(emb, idx)
    loss = jnp.sum(acc[:, 0, 0]) * ((1.0 + _BETA) / (_N_TOK * _E_DIM))
    return (z_q, loss)
